# Initial kernel scaffold; baseline (speedup 1.0000x reference)
import jax
import jax.numpy as jnp
from jax.experimental import pallas as pl


def kernel(x, TimeStage, Wg, bg, W1, b1, W2, b2):
    raise NotImplementedError("write your pallas kernel here")


# speedup vs baseline: 7.3755x; 7.3755x over previous
import functools

import jax
import jax.numpy as jnp
from jax import lax
from jax.experimental import pallas as pl
from jax.experimental.pallas import tpu as pltpu
from jax.experimental.pallas import tpu_sc as plsc

D_MODEL = 768
D_FF = 4 * D_MODEL
E = 16
N = 2048
BLK = 128
NSTEP = 32
NPAD = NSTEP * BLK
CH = 512


def _gating_body(x_ref, ts_ref, wg_ref, bg_ref,
                 dest_ref, w128_ref, se_ref, nr_ref):
    r = x_ref[...] + ts_ref[...]
    logits = jnp.dot(r, wg_ref[...], preferred_element_type=jnp.float32)
    logits = logits + bg_ref[...]

    m = jnp.max(logits, axis=1, keepdims=True)
    p = jnp.exp(logits - m)
    s = jnp.sum(p, axis=1, keepdims=True)
    w = 1.0 / s[:, 0]

    lane = lax.broadcasted_iota(jnp.int32, (N, E), 1)
    is_max = logits == m
    e_id = jnp.min(jnp.where(is_max, lane, E), axis=1)
    oh = (lane == e_id[:, None]).astype(jnp.float32)

    ri = lax.broadcasted_iota(jnp.int32, (CH, CH), 0)
    ci = lax.broadcasted_iota(jnp.int32, (CH, CH), 1)
    tri = (ci < ri).astype(jnp.float32)
    carry = jnp.zeros((1, E), jnp.float32)
    rank_chunks = []
    for c in range(N // CH):
        ohc = oh[c * CH:(c + 1) * CH, :]
        exc = jnp.dot(tri, ohc, preferred_element_type=jnp.float32) + carry
        rank_chunks.append(exc)
        carry = carry + jnp.sum(ohc, axis=0, keepdims=True)
    rank_all = jnp.concatenate(rank_chunks, axis=0)
    rank = jnp.sum(rank_all * oh, axis=1)
    counts = carry

    blocks = jnp.floor((counts + 127.0) / 128.0)
    fi = lax.broadcasted_iota(jnp.int32, (E, E), 0)
    ei = lax.broadcasted_iota(jnp.int32, (E, E), 1)
    triu = (fi < ei).astype(jnp.float32)
    bstart = jnp.dot(blocks, triu,
                     preferred_element_type=jnp.float32)
    bend = bstart + blocks

    dest = jnp.sum(oh * (128.0 * bstart), axis=1) + rank
    dest_ref[...] = dest.astype(jnp.int32).reshape(1, N)
    w128_ref[...] = jnp.broadcast_to(w[:, None], (N, 128))

    total = jnp.sum(blocks)
    step_i = lax.broadcasted_iota(jnp.int32, (NSTEP, E), 0).astype(jnp.float32)
    i_cl = jnp.minimum(step_i, total - 1.0)
    se = jnp.sum((bend <= i_cl).astype(jnp.float32), axis=1)
    lane_s = lax.broadcasted_iota(jnp.int32, (NSTEP, E), 1).astype(jnp.float32)
    oh_s = (lane_s == se[:, None]).astype(jnp.float32)
    c_sel = jnp.sum(oh_s * counts, axis=1)
    b_sel = jnp.sum(oh_s * bstart, axis=1)
    i_col = step_i[:, 0]
    nr = jnp.clip(c_sel - 128.0 * (i_col - b_sel), 0.0, 128.0)
    se_ref[...] = se.astype(jnp.int32).reshape(1, NSTEP)
    nr_ref[...] = nr.astype(jnp.int32).reshape(1, NSTEP)


def _gating(x, ts, Wg, bg):
    return pl.pallas_call(
        _gating_body,
        out_shape=[
            jax.ShapeDtypeStruct((1, N), jnp.int32),
            jax.ShapeDtypeStruct((N, 128), jnp.float32),
            jax.ShapeDtypeStruct((1, NSTEP), jnp.int32),
            jax.ShapeDtypeStruct((1, NSTEP), jnp.int32),
        ],
    )(x, ts, Wg, bg.reshape(1, E))


_NC = 2
_NS = 16
_NW = _NC * _NS
_TPW = N // _NW


@functools.cache
def _scatter_sc():
    mesh = plsc.VectorSubcoreMesh(core_axis_name="c", subcore_axis_name="s")

    @functools.partial(
        pl.kernel,
        out_type=[jax.ShapeDtypeStruct((NPAD, D_MODEL), jnp.float32),
                  jax.ShapeDtypeStruct((NPAD, 128), jnp.float32)],
        mesh=mesh,
        scratch_types=[pltpu.VMEM((_TPW,), jnp.int32),
                       pltpu.VMEM((_TPW, D_MODEL), jnp.float32),
                       pltpu.VMEM((_TPW, 128), jnp.float32),
                       pltpu.SemaphoreType.DMA],
    )
    def scatter(x_hbm, w128_hbm, dest_hbm, xpad_hbm, wpad_hbm,
                idx_v, rows_v, w_v, sem):
        wid = lax.axis_index("s") * _NC + lax.axis_index("c")
        base = wid * _TPW
        pltpu.sync_copy(dest_hbm.at[pl.ds(base, _TPW)], idx_v)
        pltpu.sync_copy(x_hbm.at[pl.ds(base, _TPW)], rows_v)
        pltpu.sync_copy(w128_hbm.at[pl.ds(base, _TPW)], w_v)
        pltpu.async_copy(rows_v, xpad_hbm.at[idx_v], sem).wait()
        pltpu.async_copy(w_v, wpad_hbm.at[idx_v], sem).wait()

    return scatter


def _ffn_body(se_ref, nr_ref, x_ref, w1_ref, b1_ref, w2_ref, b2_ref, wp_ref,
              out_ref):
    i = pl.program_id(0)
    nrows = nr_ref[i]
    ri = lax.broadcasted_iota(jnp.int32, (BLK, 1), 0)
    mask = ri < nrows
    xb = jnp.where(mask, x_ref[...], 0.0)
    h = jnp.dot(xb, w1_ref[0], preferred_element_type=jnp.float32)
    h = h + b1_ref[0]
    h = 0.5 * h * (1.0 + lax.erf(h * 0.7071067811865476))
    y = jnp.dot(h, w2_ref[0], preferred_element_type=jnp.float32)
    y = y + b2_ref[0]
    wcol = jnp.where(mask, wp_ref[:, :1], 0.0)
    out_ref[...] = y * wcol


def _ffn(se, nr, xpad, W1, b1, W2, b2, wpad):
    grid_spec = pltpu.PrefetchScalarGridSpec(
        num_scalar_prefetch=2,
        grid=(NSTEP,),
        in_specs=[
            pl.BlockSpec((BLK, D_MODEL), lambda i, se, nr: (i, 0)),
            pl.BlockSpec((1, D_MODEL, D_FF), lambda i, se, nr: (se[i], 0, 0)),
            pl.BlockSpec((1, 1, D_FF), lambda i, se, nr: (se[i], 0, 0)),
            pl.BlockSpec((1, D_FF, D_MODEL), lambda i, se, nr: (se[i], 0, 0)),
            pl.BlockSpec((1, 1, D_MODEL), lambda i, se, nr: (se[i], 0, 0)),
            pl.BlockSpec((BLK, 128), lambda i, se, nr: (i, 0)),
        ],
        out_specs=pl.BlockSpec((BLK, D_MODEL), lambda i, se, nr: (i, 0)),
    )
    return pl.pallas_call(
        _ffn_body,
        grid_spec=grid_spec,
        out_shape=jax.ShapeDtypeStruct((NPAD, D_MODEL), jnp.float32),
    )(se, nr, xpad, W1, b1.reshape(E, 1, D_FF), W2, b2.reshape(E, 1, D_MODEL),
      wpad)


@functools.cache
def _gather_sc():
    mesh = plsc.VectorSubcoreMesh(core_axis_name="c", subcore_axis_name="s")

    @functools.partial(
        pl.kernel,
        out_type=jax.ShapeDtypeStruct((N, D_MODEL), jnp.float32),
        mesh=mesh,
        scratch_types=[pltpu.VMEM((_TPW,), jnp.int32),
                       pltpu.VMEM((_TPW, D_MODEL), jnp.float32),
                       pltpu.SemaphoreType.DMA],
    )
    def gather(ypad_hbm, dest_hbm, out_hbm, idx_v, rows_v, sem):
        wid = lax.axis_index("s") * _NC + lax.axis_index("c")
        base = wid * _TPW
        pltpu.sync_copy(dest_hbm.at[pl.ds(base, _TPW)], idx_v)
        pltpu.async_copy(ypad_hbm.at[idx_v], rows_v, sem).wait()
        pltpu.sync_copy(rows_v, out_hbm.at[pl.ds(base, _TPW)])

    return gather


def kernel(x, TimeStage, Wg, bg, W1, b1, W2, b2):
    dest2d, w128, se2d, nr2d = _gating(x, TimeStage, Wg, bg)
    dest = dest2d.reshape(N)
    se = se2d.reshape(NSTEP)
    nr = nr2d.reshape(NSTEP)
    xpad, wpad = _scatter_sc()(x, w128, dest)
    ypad = _ffn(se, nr, xpad, W1, b1, W2, b2, wpad)
    return _gather_sc()(ypad, dest)


# speedup vs baseline: 7.7182x; 1.0465x over previous
import functools

import jax
import jax.numpy as jnp
from jax import lax
from jax.experimental import pallas as pl
from jax.experimental.pallas import tpu as pltpu
from jax.experimental.pallas import tpu_sc as plsc

D_MODEL = 768
D_FF = 4 * D_MODEL
E = 16
N = 2048
BLK = 128
NSTEP = 32
NPAD = NSTEP * BLK
CH = 512


def _gating_body(x_ref, ts_ref, wg_ref, bg_ref,
                 dest_ref, w128_ref, se_ref, nr_ref):
    r = x_ref[...] + ts_ref[...]
    logits = jnp.dot(r, wg_ref[...], preferred_element_type=jnp.float32)
    logits = logits + bg_ref[...]

    m = jnp.max(logits, axis=1, keepdims=True)
    p = jnp.exp(logits - m)
    s = jnp.sum(p, axis=1, keepdims=True)
    w = 1.0 / s[:, 0]

    lane = lax.broadcasted_iota(jnp.int32, (N, E), 1)
    is_max = logits == m
    e_id = jnp.min(jnp.where(is_max, lane, E), axis=1)
    oh = (lane == e_id[:, None]).astype(jnp.float32)

    ri = lax.broadcasted_iota(jnp.int32, (CH, CH), 0)
    ci = lax.broadcasted_iota(jnp.int32, (CH, CH), 1)
    tri = (ci < ri).astype(jnp.float32)
    carry = jnp.zeros((1, E), jnp.float32)
    rank_chunks = []
    for c in range(N // CH):
        ohc = oh[c * CH:(c + 1) * CH, :]
        exc = jnp.dot(tri, ohc, preferred_element_type=jnp.float32) + carry
        rank_chunks.append(exc)
        carry = carry + jnp.sum(ohc, axis=0, keepdims=True)
    rank_all = jnp.concatenate(rank_chunks, axis=0)
    rank = jnp.sum(rank_all * oh, axis=1)
    counts = carry

    blocks = jnp.floor((counts + 127.0) / 128.0)
    fi = lax.broadcasted_iota(jnp.int32, (E, E), 0)
    ei = lax.broadcasted_iota(jnp.int32, (E, E), 1)
    triu = (fi < ei).astype(jnp.float32)
    bstart = jnp.dot(blocks, triu,
                     preferred_element_type=jnp.float32)
    bend = bstart + blocks

    dest = jnp.sum(oh * (128.0 * bstart), axis=1) + rank
    dest_ref[...] = dest.astype(jnp.int32).reshape(1, N)
    w128_ref[...] = jnp.broadcast_to(w[:, None], (N, 128))

    total = jnp.sum(blocks)
    step_i = lax.broadcasted_iota(jnp.int32, (NSTEP, E), 0).astype(jnp.float32)
    i_cl = jnp.minimum(step_i, total - 1.0)
    se = jnp.sum((bend <= i_cl).astype(jnp.float32), axis=1)
    lane_s = lax.broadcasted_iota(jnp.int32, (NSTEP, E), 1).astype(jnp.float32)
    oh_s = (lane_s == se[:, None]).astype(jnp.float32)
    c_sel = jnp.sum(oh_s * counts, axis=1)
    b_sel = jnp.sum(oh_s * bstart, axis=1)
    i_col = step_i[:, 0]
    nr = jnp.clip(c_sel - 128.0 * (i_col - b_sel), 0.0, 128.0)
    se_ref[...] = se.astype(jnp.int32).reshape(1, NSTEP)
    nr_ref[...] = nr.astype(jnp.int32).reshape(1, NSTEP)


def _gating(x, ts, Wg, bg):
    return pl.pallas_call(
        _gating_body,
        out_shape=[
            jax.ShapeDtypeStruct((1, N), jnp.int32),
            jax.ShapeDtypeStruct((N, 128), jnp.float32),
            jax.ShapeDtypeStruct((1, NSTEP), jnp.int32),
            jax.ShapeDtypeStruct((1, NSTEP), jnp.int32),
        ],
    )(x, ts, Wg, bg.reshape(1, E))


_NC = 2
_NS = 16
_NW = _NC * _NS
_TPW = N // _NW


@functools.cache
def _scatter_sc():
    mesh = plsc.VectorSubcoreMesh(core_axis_name="c", subcore_axis_name="s")

    @functools.partial(
        pl.kernel,
        out_type=[jax.ShapeDtypeStruct((NPAD, D_MODEL), jnp.float32),
                  jax.ShapeDtypeStruct((NPAD, 128), jnp.float32)],
        mesh=mesh,
        scratch_types=[pltpu.VMEM((_TPW,), jnp.int32),
                       pltpu.VMEM((_TPW, D_MODEL), jnp.float32),
                       pltpu.VMEM((_TPW, 128), jnp.float32),
                       pltpu.SemaphoreType.DMA],
    )
    def scatter(x_hbm, w128_hbm, dest_hbm, xpad_hbm, wpad_hbm,
                idx_v, rows_v, w_v, sem):
        wid = lax.axis_index("s") * _NC + lax.axis_index("c")
        base = wid * _TPW
        pltpu.sync_copy(dest_hbm.at[pl.ds(base, _TPW)], idx_v)
        pltpu.sync_copy(x_hbm.at[pl.ds(base, _TPW)], rows_v)
        pltpu.sync_copy(w128_hbm.at[pl.ds(base, _TPW)], w_v)
        pltpu.async_copy(rows_v, xpad_hbm.at[idx_v], sem).wait()
        pltpu.async_copy(w_v, wpad_hbm.at[idx_v], sem).wait()

    return scatter


def _ffn_body(se_ref, nr_ref, x_ref, w1_ref, b1_ref, w2_ref, b2_ref, wp_ref,
              out_ref):
    i = pl.program_id(0)
    nrows = nr_ref[i]

    @pl.when(nrows > 0)
    def _():
        ri = lax.broadcasted_iota(jnp.int32, (BLK, 1), 0)
        mask = ri < nrows
        xb = jnp.where(mask, x_ref[...], 0.0)
        h = jnp.dot(xb, w1_ref[0], preferred_element_type=jnp.float32)
        h = h + b1_ref[0]
        h = 0.5 * h * (1.0 + lax.erf(h * 0.7071067811865476))
        y = jnp.dot(h, w2_ref[0], preferred_element_type=jnp.float32)
        y = y + b2_ref[0]
        wcol = jnp.where(mask, wp_ref[:, :1], 0.0)
        out_ref[...] = y * wcol


def _ffn(se, nr, xpad, W1, b1, W2, b2, wpad):
    grid_spec = pltpu.PrefetchScalarGridSpec(
        num_scalar_prefetch=2,
        grid=(NSTEP,),
        in_specs=[
            pl.BlockSpec((BLK, D_MODEL), lambda i, se, nr: (i, 0)),
            pl.BlockSpec((1, D_MODEL, D_FF), lambda i, se, nr: (se[i], 0, 0)),
            pl.BlockSpec((1, 1, D_FF), lambda i, se, nr: (se[i], 0, 0)),
            pl.BlockSpec((1, D_FF, D_MODEL), lambda i, se, nr: (se[i], 0, 0)),
            pl.BlockSpec((1, 1, D_MODEL), lambda i, se, nr: (se[i], 0, 0)),
            pl.BlockSpec((BLK, 128), lambda i, se, nr: (i, 0)),
        ],
        out_specs=pl.BlockSpec((BLK, D_MODEL), lambda i, se, nr: (i, 0)),
    )
    return pl.pallas_call(
        _ffn_body,
        grid_spec=grid_spec,
        out_shape=jax.ShapeDtypeStruct((NPAD, D_MODEL), jnp.float32),
    )(se, nr, xpad, W1, b1.reshape(E, 1, D_FF), W2, b2.reshape(E, 1, D_MODEL),
      wpad)


@functools.cache
def _gather_sc():
    mesh = plsc.VectorSubcoreMesh(core_axis_name="c", subcore_axis_name="s")

    @functools.partial(
        pl.kernel,
        out_type=jax.ShapeDtypeStruct((N, D_MODEL), jnp.float32),
        mesh=mesh,
        scratch_types=[pltpu.VMEM((_TPW,), jnp.int32),
                       pltpu.VMEM((_TPW, D_MODEL), jnp.float32),
                       pltpu.SemaphoreType.DMA],
    )
    def gather(ypad_hbm, dest_hbm, out_hbm, idx_v, rows_v, sem):
        wid = lax.axis_index("s") * _NC + lax.axis_index("c")
        base = wid * _TPW
        pltpu.sync_copy(dest_hbm.at[pl.ds(base, _TPW)], idx_v)
        pltpu.async_copy(ypad_hbm.at[idx_v], rows_v, sem).wait()
        pltpu.sync_copy(rows_v, out_hbm.at[pl.ds(base, _TPW)])

    return gather


def kernel(x, TimeStage, Wg, bg, W1, b1, W2, b2):
    dest2d, w128, se2d, nr2d = _gating(x, TimeStage, Wg, bg)
    dest = dest2d.reshape(N)
    se = se2d.reshape(NSTEP)
    nr = nr2d.reshape(NSTEP)
    xpad, wpad = _scatter_sc()(x, w128, dest)
    ypad = _ffn(se, nr, xpad, W1, b1, W2, b2, wpad)
    return _gather_sc()(ypad, dest)
